# baseline (speedup 1.0000x reference)
import functools

import jax
import jax.numpy as jnp
from jax import lax
from jax.experimental import pallas as pl
from jax.experimental.pallas import tpu as pltpu
from jax.experimental.pallas import tpu_sc as plsc

VOCAB = 1000000
EMBED = 32
BATCH = 16384
HIST = 200


_S_BLK = 32768
_S_GRID = (VOCAB + _S_BLK - 1) // _S_BLK


def _s_body(w_ref, t_ref, o_ref):
    o = lax.dot_general(
        w_ref[...], t_ref[...],
        dimension_numbers=(((1,), (0,)), ((), ())),
        preferred_element_type=jnp.float32,
    )
    o_ref[...] = o[0]


def _compute_s(table_t, w_scaled):
    w8 = jnp.broadcast_to(w_scaled, (8, EMBED))
    return pl.pallas_call(
        _s_body,
        grid=(_S_GRID,),
        in_specs=[
            pl.BlockSpec((8, EMBED), lambda i: (0, 0)),
            pl.BlockSpec((EMBED, _S_BLK), lambda i: (0, i)),
        ],
        out_specs=pl.BlockSpec((_S_BLK,), lambda i: (i,)),
        out_shape=jax.ShapeDtypeStruct((VOCAB,), jnp.float32),
    )(w8, table_t)


_NC = 2
_NS = 16
_NW = _NC * _NS
_ROWS_W = BATCH // _NW
_GRP = 16
_NGRP = _ROWS_W // _GRP

_GSZ = HIST * _GRP


@functools.cache
def _make_sc_transpose():
    mesh = plsc.VectorSubcoreMesh(core_axis_name="c", subcore_axis_name="s")
    ngrp_tot = BATCH // _GRP

    @functools.partial(
        pl.kernel,
        out_type=jax.ShapeDtypeStruct((ngrp_tot, HIST, _GRP), jnp.int32),
        mesh=mesh,
        scratch_types=[
            pltpu.VMEM((2, HIST, _GRP), jnp.int32),
            pltpu.SemaphoreType.DMA,
            pltpu.SemaphoreType.DMA,
        ],
    )
    def _sc_t(xt_hbm, xg_hbm, buf_v, sem_i, sem_o):
        wid = lax.axis_index("s") * _NC + lax.axis_index("c")
        gbase = wid * _NGRP
        pltpu.async_copy(xt_hbm.at[:, gbase, :], buf_v.at[0], sem_i).wait()

        def pair(h, carry):
            g0 = gbase + 2 * h
            g1 = g0 + 1
            pltpu.async_copy(xt_hbm.at[:, g1, :], buf_v.at[1], sem_i)
            pltpu.async_copy(buf_v.at[0], xg_hbm.at[g0], sem_o).wait()
            pltpu.make_async_copy(xt_hbm.at[:, g1, :], buf_v.at[1], sem_i).wait()

            @pl.when(h + 1 < _NGRP // 2)
            def _():
                pltpu.async_copy(xt_hbm.at[:, g0 + 2, :], buf_v.at[0], sem_i)

            pltpu.async_copy(buf_v.at[1], xg_hbm.at[g1], sem_o).wait()

            @pl.when(h + 1 < _NGRP // 2)
            def _():
                pltpu.make_async_copy(
                    xt_hbm.at[:, g0 + 2, :], buf_v.at[0], sem_i
                ).wait()

            return carry

        lax.fori_loop(0, _NGRP // 2, pair, 0)

    return _sc_t


@functools.cache
def _make_sc_pool():
    mesh = plsc.VectorSubcoreMesh(core_axis_name="c", subcore_axis_name="s")

    @functools.partial(
        pl.kernel,
        out_type=jax.ShapeDtypeStruct((BATCH,), jnp.float32),
        mesh=mesh,
        scratch_types=[
            pltpu.VMEM((_GSZ,), jnp.int32),
            pltpu.VMEM((_GSZ,), jnp.float32),
            pltpu.VMEM((_GRP,), jnp.float32),
            pltpu.SemaphoreType.DMA,
        ],
    )
    def _sc_pool(xg_hbm, s_hbm, bv_hbm, out_hbm, idx_v, vals_v, out_v, sem):
        wid = lax.axis_index("s") * _NC + lax.axis_index("c")
        gbase = wid * _NGRP
        pltpu.sync_copy(bv_hbm, out_v)
        bv = out_v[...]

        def group(g, carry):
            ggl = gbase + g
            pltpu.sync_copy(xg_hbm.at[ggl], idx_v)
            pltpu.async_copy(s_hbm.at[idx_v], vals_v, sem).wait()

            def accum(l, a):
                return a + vals_v[pl.ds(l * _GRP, _GRP)]

            acc = lax.fori_loop(0, HIST, accum, bv)
            out_v[...] = 1.0 / (1.0 + jnp.exp(-acc))
            pltpu.sync_copy(out_v, out_hbm.at[pl.ds(ggl * _GRP, _GRP)])
            return carry

        lax.fori_loop(0, _NGRP, group, 0)

    return _sc_pool


def kernel(x, table, W, b):
    w_scaled = (W / HIST).astype(jnp.float32)
    s = _compute_s(table.T, w_scaled)
    xt3 = x.astype(jnp.int32).T.reshape(HIST, BATCH // _GRP, _GRP)
    xg3 = _make_sc_transpose()(xt3)
    xg = xg3.reshape(BATCH // _GRP, _GSZ)
    bv = jnp.broadcast_to(b.astype(jnp.float32), (_GRP,))
    return _make_sc_pool()(xg, s, bv)


# speedup vs baseline: 1.5763x; 1.5763x over previous
import functools

import jax
import jax.numpy as jnp
from jax import lax
from jax.experimental import pallas as pl
from jax.experimental.pallas import tpu as pltpu
from jax.experimental.pallas import tpu_sc as plsc

VOCAB = 1000000
EMBED = 32
BATCH = 16384
HIST = 200


_S_BLK = 32768
_S_GRID = (VOCAB + _S_BLK - 1) // _S_BLK


def _s_body(w_ref, t_ref, o_ref):
    o = lax.dot_general(
        w_ref[...], t_ref[...],
        dimension_numbers=(((1,), (0,)), ((), ())),
        preferred_element_type=jnp.float32,
    )
    o_ref[...] = o[0]


def _compute_s(table_t, w_scaled):
    w8 = jnp.broadcast_to(w_scaled, (8, EMBED))
    return pl.pallas_call(
        _s_body,
        grid=(_S_GRID,),
        in_specs=[
            pl.BlockSpec((8, EMBED), lambda i: (0, 0)),
            pl.BlockSpec((EMBED, _S_BLK), lambda i: (0, i)),
        ],
        out_specs=pl.BlockSpec((_S_BLK,), lambda i: (i,)),
        out_shape=jax.ShapeDtypeStruct((VOCAB,), jnp.float32),
    )(w8, table_t)


_T_GBLK = 8


def _t_body(x_ref, o_ref):
    blk = x_ref[...]
    o_ref[...] = (
        blk.reshape(HIST, _T_GBLK, 16).transpose(1, 0, 2).reshape(_T_GBLK, 16 * HIST)
    )


def _transpose_groups(x_t):
    ngrp = BATCH // 16
    return pl.pallas_call(
        _t_body,
        grid=(ngrp // _T_GBLK,),
        in_specs=[pl.BlockSpec((HIST, _T_GBLK * 16), lambda i: (0, i))],
        out_specs=pl.BlockSpec((_T_GBLK, 16 * HIST), lambda i: (i, 0)),
        out_shape=jax.ShapeDtypeStruct((ngrp, 16 * HIST), jnp.int32),
    )(x_t)


_NC = 2
_NS = 16
_NW = _NC * _NS
_ROWS_W = BATCH // _NW
_GRP = 16
_NGRP = _ROWS_W // _GRP

_GSZ = HIST * _GRP


@functools.cache
def _make_sc_pool():
    mesh = plsc.VectorSubcoreMesh(core_axis_name="c", subcore_axis_name="s")

    @functools.partial(
        pl.kernel,
        out_type=jax.ShapeDtypeStruct((BATCH,), jnp.float32),
        mesh=mesh,
        scratch_types=[
            pltpu.VMEM((_GSZ,), jnp.int32),
            pltpu.VMEM((_GSZ,), jnp.int32),
            pltpu.VMEM((_GSZ,), jnp.float32),
            pltpu.VMEM((_GSZ,), jnp.float32),
            pltpu.VMEM((_GRP,), jnp.float32),
            pltpu.SemaphoreType.DMA,
            pltpu.SemaphoreType.DMA,
        ],
    )
    def _sc_pool(
        xg_hbm, s_hbm, bv_hbm, out_hbm, idx0, idx1, val0, val1, out_v, s0, s1
    ):
        wid = lax.axis_index("s") * _NC + lax.axis_index("c")
        gbase = wid * _NGRP
        pltpu.sync_copy(bv_hbm, out_v)
        bv = out_v[...]
        bufs = ((idx0, val0, s0), (idx1, val1, s1))

        def start(b, ggl):
            idx_v, vals_v, sem = bufs[b]
            pltpu.sync_copy(xg_hbm.at[ggl], idx_v)
            pltpu.async_copy(s_hbm.at[idx_v], vals_v, sem)

        def finish(b, ggl):
            idx_v, vals_v, sem = bufs[b]
            pltpu.make_async_copy(s_hbm.at[idx_v], vals_v, sem).wait()

            def accum(l, a):
                return a + vals_v[pl.ds(l * _GRP, _GRP)]

            acc = lax.fori_loop(0, HIST, accum, bv)
            out_v[...] = 1.0 / (1.0 + jnp.exp(-acc))
            pltpu.sync_copy(out_v, out_hbm.at[pl.ds(ggl * _GRP, _GRP)])

        start(0, gbase)

        def pair(h, carry):
            g0 = gbase + 2 * h
            start(1, g0 + 1)
            finish(0, g0)

            @pl.when(h + 1 < _NGRP // 2)
            def _():
                start(0, g0 + 2)

            finish(1, g0 + 1)
            return carry

        lax.fori_loop(0, _NGRP // 2, pair, 0)

    return _sc_pool


def kernel(x, table, W, b):
    w_scaled = (W / HIST).astype(jnp.float32)
    s = _compute_s(table.T, w_scaled)
    xg = _transpose_groups(x.astype(jnp.int32).T)
    bv = jnp.broadcast_to(b.astype(jnp.float32), (_GRP,))
    return _make_sc_pool()(xg, s, bv)


# speedup vs baseline: 1.8188x; 1.1539x over previous
import functools

import jax
import jax.numpy as jnp
from jax import lax
from jax.experimental import pallas as pl
from jax.experimental.pallas import tpu as pltpu
from jax.experimental.pallas import tpu_sc as plsc

VOCAB = 1000000
EMBED = 32
BATCH = 16384
HIST = 200


_S_BLK = 32768
_S_GRID = (VOCAB + _S_BLK - 1) // _S_BLK


def _s_body(w_ref, t_ref, o_ref):
    o = lax.dot_general(
        w_ref[...], t_ref[...],
        dimension_numbers=(((1,), (0,)), ((), ())),
        preferred_element_type=jnp.float32,
    )
    o_ref[...] = o[0]


def _compute_s(table_t, w_scaled):
    w8 = jnp.broadcast_to(w_scaled, (8, EMBED))
    return pl.pallas_call(
        _s_body,
        grid=(_S_GRID,),
        in_specs=[
            pl.BlockSpec((8, EMBED), lambda i: (0, 0)),
            pl.BlockSpec((EMBED, _S_BLK), lambda i: (0, i)),
        ],
        out_specs=pl.BlockSpec((_S_BLK,), lambda i: (i,)),
        out_shape=jax.ShapeDtypeStruct((VOCAB,), jnp.float32),
    )(w8, table_t)


_T_GBLK = 32


def _t_body(x_ref, o_ref):
    blk = x_ref[...]
    o_ref[...] = (
        blk.reshape(HIST, _T_GBLK, 16).transpose(1, 0, 2).reshape(_T_GBLK, 16 * HIST)
    )


def _transpose_groups(x_t):
    ngrp = BATCH // 16
    return pl.pallas_call(
        _t_body,
        grid=(ngrp // _T_GBLK,),
        in_specs=[pl.BlockSpec((HIST, _T_GBLK * 16), lambda i: (0, i))],
        out_specs=pl.BlockSpec((_T_GBLK, 16 * HIST), lambda i: (i, 0)),
        out_shape=jax.ShapeDtypeStruct((ngrp, 16 * HIST), jnp.int32),
    )(x_t)


_NC = 2
_NS = 16
_NW = _NC * _NS
_ROWS_W = BATCH // _NW
_GRP = 16
_NGRP = _ROWS_W // _GRP

_GSZ = HIST * _GRP


@functools.cache
def _make_sc_pool():
    mesh = plsc.VectorSubcoreMesh(core_axis_name="c", subcore_axis_name="s")

    @functools.partial(
        pl.kernel,
        out_type=jax.ShapeDtypeStruct((BATCH,), jnp.float32),
        mesh=mesh,
        scratch_types=[
            pltpu.VMEM((_GSZ,), jnp.int32),
            pltpu.VMEM((_GSZ,), jnp.int32),
            pltpu.VMEM((_GSZ,), jnp.float32),
            pltpu.VMEM((_GSZ,), jnp.float32),
            pltpu.VMEM((_GRP,), jnp.float32),
            pltpu.SemaphoreType.DMA,
            pltpu.SemaphoreType.DMA,
        ],
    )
    def _sc_pool(
        xg_hbm, s_hbm, bv_hbm, out_hbm, idx0, idx1, val0, val1, out_v, s0, s1
    ):
        wid = lax.axis_index("s") * _NC + lax.axis_index("c")
        gbase = wid * _NGRP
        pltpu.sync_copy(bv_hbm, out_v)
        bv = out_v[...]
        bufs = ((idx0, val0, s0), (idx1, val1, s1))

        def start(b, ggl):
            idx_v, vals_v, sem = bufs[b]
            pltpu.sync_copy(xg_hbm.at[ggl], idx_v)
            pltpu.async_copy(s_hbm.at[idx_v], vals_v, sem)

        def finish(b, ggl):
            idx_v, vals_v, sem = bufs[b]
            pltpu.make_async_copy(s_hbm.at[idx_v], vals_v, sem).wait()

            def accum(l, a):
                return a + vals_v[pl.ds(l * _GRP, _GRP)]

            acc = lax.fori_loop(0, HIST, accum, bv)
            out_v[...] = 1.0 / (1.0 + jnp.exp(-acc))
            pltpu.sync_copy(out_v, out_hbm.at[pl.ds(ggl * _GRP, _GRP)])

        start(0, gbase)

        def pair(h, carry):
            g0 = gbase + 2 * h
            start(1, g0 + 1)
            finish(0, g0)

            @pl.when(h + 1 < _NGRP // 2)
            def _():
                start(0, g0 + 2)

            finish(1, g0 + 1)
            return carry

        lax.fori_loop(0, _NGRP // 2, pair, 0)

    return _sc_pool


def kernel(x, table, W, b):
    w_scaled = (W / HIST).astype(jnp.float32)
    s = _compute_s(table.T, w_scaled)
    xg = _transpose_groups(x.astype(jnp.int32).T)
    bv = jnp.broadcast_to(b.astype(jnp.float32), (_GRP,))
    return _make_sc_pool()(xg, s, bv)


# speedup vs baseline: 1.8318x; 1.0072x over previous
import functools

import jax
import jax.numpy as jnp
from jax import lax
from jax.experimental import pallas as pl
from jax.experimental.pallas import tpu as pltpu
from jax.experimental.pallas import tpu_sc as plsc

VOCAB = 1000000
EMBED = 32
BATCH = 16384
HIST = 200


_S_BLK = 32768
_S_GRID = (VOCAB + _S_BLK - 1) // _S_BLK


def _s_body(w_ref, t_ref, o_ref):
    o = lax.dot_general(
        w_ref[...], t_ref[...],
        dimension_numbers=(((1,), (0,)), ((), ())),
        preferred_element_type=jnp.float32,
    )
    o_ref[...] = o[0]


def _compute_s(table_t, w_scaled):
    w8 = jnp.broadcast_to(w_scaled, (8, EMBED))
    return pl.pallas_call(
        _s_body,
        grid=(_S_GRID,),
        in_specs=[
            pl.BlockSpec((8, EMBED), lambda i: (0, 0)),
            pl.BlockSpec((EMBED, _S_BLK), lambda i: (0, i)),
        ],
        out_specs=pl.BlockSpec((_S_BLK,), lambda i: (i,)),
        out_shape=jax.ShapeDtypeStruct((VOCAB,), jnp.float32),
    )(w8, table_t)


_T_GBLK = 64


def _t_body(x_ref, o_ref):
    blk = x_ref[...]
    o_ref[...] = (
        blk.reshape(HIST, _T_GBLK, 16).transpose(1, 0, 2).reshape(_T_GBLK, 16 * HIST)
    )


def _transpose_groups(x_t):
    ngrp = BATCH // 16
    return pl.pallas_call(
        _t_body,
        grid=(ngrp // _T_GBLK,),
        in_specs=[pl.BlockSpec((HIST, _T_GBLK * 16), lambda i: (0, i))],
        out_specs=pl.BlockSpec((_T_GBLK, 16 * HIST), lambda i: (i, 0)),
        out_shape=jax.ShapeDtypeStruct((ngrp, 16 * HIST), jnp.int32),
    )(x_t)


_NC = 2
_NS = 16
_NW = _NC * _NS
_ROWS_W = BATCH // _NW
_GRP = 16
_NGRP = _ROWS_W // _GRP

_GSZ = HIST * _GRP


@functools.cache
def _make_sc_pool():
    mesh = plsc.VectorSubcoreMesh(core_axis_name="c", subcore_axis_name="s")

    @functools.partial(
        pl.kernel,
        out_type=jax.ShapeDtypeStruct((BATCH,), jnp.float32),
        mesh=mesh,
        scratch_types=[
            pltpu.VMEM((_GSZ,), jnp.int32),
            pltpu.VMEM((_GSZ,), jnp.int32),
            pltpu.VMEM((_GSZ,), jnp.float32),
            pltpu.VMEM((_GSZ,), jnp.float32),
            pltpu.VMEM((_GRP,), jnp.float32),
            pltpu.SemaphoreType.DMA,
            pltpu.SemaphoreType.DMA,
        ],
    )
    def _sc_pool(
        xg_hbm, s_hbm, bv_hbm, out_hbm, idx0, idx1, val0, val1, out_v, s0, s1
    ):
        wid = lax.axis_index("s") * _NC + lax.axis_index("c")
        gbase = wid * _NGRP
        pltpu.sync_copy(bv_hbm, out_v)
        bv = out_v[...]
        bufs = ((idx0, val0, s0), (idx1, val1, s1))

        def start(b, ggl):
            idx_v, vals_v, sem = bufs[b]
            pltpu.sync_copy(xg_hbm.at[ggl], idx_v)
            pltpu.async_copy(s_hbm.at[idx_v], vals_v, sem)

        def finish(b, ggl):
            idx_v, vals_v, sem = bufs[b]
            pltpu.make_async_copy(s_hbm.at[idx_v], vals_v, sem).wait()

            def accum(l, a):
                return a + vals_v[pl.ds(l * _GRP, _GRP)]

            acc = lax.fori_loop(0, HIST, accum, bv)
            out_v[...] = 1.0 / (1.0 + jnp.exp(-acc))
            pltpu.sync_copy(out_v, out_hbm.at[pl.ds(ggl * _GRP, _GRP)])

        start(0, gbase)

        def pair(h, carry):
            g0 = gbase + 2 * h
            start(1, g0 + 1)
            finish(0, g0)

            @pl.when(h + 1 < _NGRP // 2)
            def _():
                start(0, g0 + 2)

            finish(1, g0 + 1)
            return carry

        lax.fori_loop(0, _NGRP // 2, pair, 0)

    return _sc_pool


def kernel(x, table, W, b):
    w_scaled = (W / HIST).astype(jnp.float32)
    s = _compute_s(table.T, w_scaled)
    xg = _transpose_groups(x.astype(jnp.int32).T)
    bv = jnp.broadcast_to(b.astype(jnp.float32), (_GRP,))
    return _make_sc_pool()(xg, s, bv)
